# baseline (speedup 1.0000x reference)
import functools

import jax
import jax.numpy as jnp
from jax import lax
from jax.experimental import pallas as pl
from jax.experimental.pallas import tpu as pltpu
from jax.experimental.pallas import tpu_sc as plsc

_N = 10000
_E = 320000
_D = 128

_NC = 2
_NS = 16

_CH = 128
_C = 160
_EPAD = _NS * _C * _CH
_NBUF = 2
_SS = 2
_SW = _CH // _SS
_RB = 128
_NPAD = 10240
_NH = _NPAD // _NC
_NHB = _NH // _RB
_AROWS = _NH
_DSPLIT = (_NHB - 1) * _RB


def _deg_body(idxc, ones_hbm, zeros_hbm, deg_out,
              idx_v, ones_v, zeros_v, acc):
    cid = lax.axis_index("c")
    sid = lax.axis_index("s")

    pltpu.sync_copy(ones_hbm, ones_v)
    pltpu.sync_copy(zeros_hbm, zeros_v)

    def zloop(k, carry):
        b = sid + _NS * k

        @pl.when(b < _NHB)
        def _z():
            pltpu.sync_copy(zeros_v, acc.at[pl.ds(b * _CH, _CH)])
        return carry

    lax.fori_loop(0, _NHB // _NS + 1, zloop, 0)
    pltpu.sync_copy(idxc.at[cid].at[sid], idx_v)
    plsc.subcore_barrier()

    def eloop(j, carry):
        pltpu.sync_copy(ones_v, acc.at[idx_v.at[j]], add=True)
        return carry

    lax.fori_loop(0, _C, eloop, 0)
    plsc.subcore_barrier()

    def oloop(k, carry):
        b = sid + _NS * k

        @pl.when(b < _NHB)
        def _o():
            pltpu.sync_copy(acc.at[pl.ds(b * _CH, _CH)],
                            deg_out.at[cid].at[pl.ds(b * _CH, _CH)])
        return carry

    lax.fori_loop(0, _NHB // _NS + 1, oloop, 0)


@functools.cache
def _deg_call():
    mesh = plsc.VectorSubcoreMesh(
        core_axis_name="c", subcore_axis_name="s",
        num_cores=_NC, num_subcores=_NS)
    return pl.kernel(
        _deg_body,
        out_type=jax.ShapeDtypeStruct((_NC, _NH, _D), jnp.float32),
        mesh=mesh,
        scratch_types=[
            pltpu.VMEM((_C, _CH), jnp.int32),
            pltpu.VMEM((_CH, _D), jnp.float32),
            pltpu.VMEM((_CH, _D), jnp.float32),
            pltpu.VMEM_SHARED((_NH, _D), jnp.float32),
        ],
    )


def _agg_body(h_hbm, srcb, dstc, out_hbm,
              src_v, dst_v, rows_v, acc, *sems):
    cid = lax.axis_index("c")
    sid = lax.axis_index("s")

    pltpu.sync_copy(h_hbm.at[pl.ds(_NPAD - _RB, _RB)], rows_v.at[0])

    def zloop(k, carry):
        b = sid + _NS * k

        @pl.when(b < _NHB)
        def _z():
            pltpu.sync_copy(rows_v.at[0], acc.at[pl.ds(b * _RB, _RB)])
        return carry

    lax.fori_loop(0, _NHB // _NS + 1, zloop, 0)
    pltpu.sync_copy(srcb.at[cid].at[sid], src_v)
    pltpu.sync_copy(dstc.at[cid].at[sid], dst_v)
    plsc.subcore_barrier()

    def _fire(j, slot, sem):
        for s in range(_SS):
            pltpu.async_copy(
                h_hbm.at[src_v.at[j].at[pl.ds(s * _SW, _SW)]],
                rows_v.at[slot].at[pl.ds(s * _SW, _SW)], sem)

    def _drain(slot, sem):
        pltpu.make_async_copy(
            h_hbm.at[pl.ds(0, _CH)], rows_v.at[slot], sem).wait()

    for r in range(_NBUF):
        _fire(r, r, sems[r])

    def gloop(g, carry):
        for r in range(_NBUF):
            j = _NBUF * g + r
            _drain(r, sems[r])
            pltpu.sync_copy(rows_v.at[r], acc.at[dst_v.at[j]], add=True)

            @pl.when(j + _NBUF < _C)
            def _next():
                _fire(j + _NBUF, r, sems[r])
        return carry

    lax.fori_loop(0, _C // _NBUF, gloop, 0)
    plsc.subcore_barrier()

    def oloop(k, carry):
        b = sid + _NS * k

        @pl.when(b < _NHB)
        def _o():
            pltpu.sync_copy(acc.at[pl.ds(b * _CH, _CH)],
                            out_hbm.at[cid].at[pl.ds(b * _CH, _CH)])
        return carry

    lax.fori_loop(0, _NHB // _NS + 1, oloop, 0)


@functools.cache
def _agg_call():
    mesh = plsc.VectorSubcoreMesh(
        core_axis_name="c", subcore_axis_name="s",
        num_cores=_NC, num_subcores=_NS)
    return pl.kernel(
        _agg_body,
        out_type=jax.ShapeDtypeStruct((_NC, _NH, _D), jnp.float32),
        mesh=mesh,
        scratch_types=[
            pltpu.VMEM((_C, _CH), jnp.int32),
            pltpu.VMEM((_C, _CH), jnp.int32),
            pltpu.VMEM((_NBUF, _CH, _D), jnp.float32),
            pltpu.VMEM_SHARED((_AROWS, _D), jnp.float32),
        ] + [pltpu.SemaphoreType.DMA] * _NBUF,
    )


def _deg_block(i):
    c = jnp.where(i < _NHB - 1, 1, 0)
    r = jnp.where(i < _NHB - 1, i, jnp.minimum(i - (_NHB - 1), _NHB - 1))
    return (c, r, 0)


def _mm1_body(x_ref, w_ref, od_ref, id_ref, h_ref, ns_ref, nd_ref):
    odeg = od_ref[0][:, :1]
    ideg = id_ref[0][:, :1]
    ns = lax.rsqrt(jnp.maximum(odeg, 1.0))
    nd = lax.rsqrt(jnp.maximum(ideg, 1.0))
    h = jnp.dot(x_ref[...], w_ref[...], preferred_element_type=jnp.float32)
    h_ref[...] = h * ns
    ns_ref[...] = ns
    nd_ref[...] = nd


_mm1 = pl.pallas_call(
    _mm1_body,
    grid=(_NPAD // _RB,),
    in_specs=[
        pl.BlockSpec((_RB, _D), lambda i: (i, 0)),
        pl.BlockSpec((_D, _D), lambda i: (0, 0)),
        pl.BlockSpec((1, _RB, _D), _deg_block),
        pl.BlockSpec((1, _RB, _D), _deg_block),
    ],
    out_specs=[
        pl.BlockSpec((_RB, _D), lambda i: (i, 0)),
        pl.BlockSpec((_RB, 1), lambda i: (i, 0)),
        pl.BlockSpec((_RB, 1), lambda i: (i, 0)),
    ],
    out_shape=[
        jax.ShapeDtypeStruct((_NPAD, _D), jnp.float32),
        jax.ShapeDtypeStruct((_NPAD, 1), jnp.float32),
        jax.ShapeDtypeStruct((_NPAD, 1), jnp.float32),
    ],
)


def _mm2_body(p_ref, nd_ref, b1_ref, ns_ref, w_ref, out_ref):
    i = pl.program_id(0)
    agg = p_ref[0]
    h = jnp.tanh(agg * nd_ref[...] + b1_ref[...])
    rows = i * _RB + lax.broadcasted_iota(jnp.int32, (_RB, 1), 0)
    h = jnp.where(rows < _N, h, 0.0)
    out_ref[...] = jnp.dot(h * ns_ref[...], w_ref[...],
                           preferred_element_type=jnp.float32)


_mm2 = pl.pallas_call(
    _mm2_body,
    grid=(_NPAD // _RB,),
    in_specs=[
        pl.BlockSpec((1, _RB, _D), lambda i: (i // _NHB, i % _NHB, 0)),
        pl.BlockSpec((_RB, 1), lambda i: (i, 0)),
        pl.BlockSpec((_D,), lambda i: (0,)),
        pl.BlockSpec((_RB, 1), lambda i: (i, 0)),
        pl.BlockSpec((_D, _D), lambda i: (0, 0)),
    ],
    out_specs=pl.BlockSpec((_RB, _D), lambda i: (i, 0)),
    out_shape=jax.ShapeDtypeStruct((_NPAD, _D), jnp.float32),
)


def _mm3_body(p_ref, nd_ref, b2_ref, out_ref):
    out_ref[...] = p_ref[0] * nd_ref[...] + b2_ref[...]


_mm3 = pl.pallas_call(
    _mm3_body,
    grid=(_NPAD // _RB,),
    in_specs=[
        pl.BlockSpec((1, _RB, _D), lambda i: (i // _NHB, i % _NHB, 0)),
        pl.BlockSpec((_RB, 1), lambda i: (i, 0)),
        pl.BlockSpec((_D,), lambda i: (0,)),
    ],
    out_specs=pl.BlockSpec((_RB, _D), lambda i: (i, 0)),
    out_shape=jax.ShapeDtypeStruct((_NPAD, _D), jnp.float32),
)


def kernel(graph, x, W1, b1, W2, b2):
    src = graph[0].astype(jnp.int32)
    dst = graph[1].astype(jnp.int32)
    padv = jnp.full((_EPAD - _E,), _N, jnp.int32)
    dstp = jnp.concatenate([dst, padv])
    srcp = jnp.concatenate([src, padv])

    def _local(idx, c):
        return idx - c * _NH

    def _owned(idx, c):
        loc = _local(idx, c)
        return (loc >= 0) & (loc < _NH)

    src_halves, dst_halves = [], []
    for c in range(_NC):
        own_d = _owned(dstp, c)
        src_halves.append(
            jnp.where(own_d, srcp, _N).reshape(_NS, _C, _CH))
        dst_halves.append(
            jnp.where(own_d, _local(dstp, c), 0).reshape(_NS, _C, _CH))
    srcc = jnp.stack(src_halves)
    dstc = jnp.stack(dst_halves)

    def _deg_idx(idx):
        c0 = jnp.where(idx >= _DSPLIT, idx - _DSPLIT, _NH - 1)
        c1 = jnp.where(idx < _DSPLIT, idx, _DSPLIT)
        return jnp.stack([c0.reshape(_NS, _C, _CH),
                          c1.reshape(_NS, _C, _CH)])

    sdegc = _deg_idx(srcp)
    ddegc = _deg_idx(dstp)
    xp = jnp.pad(x, ((0, _NPAD - _N), (0, 0)))
    onesf = jnp.ones((_CH, _D), jnp.float32)
    zerosf = jnp.zeros((_CH, _D), jnp.float32)

    od = _deg_call()(sdegc, onesf, zerosf)
    idg = _deg_call()(ddegc, onesf, zerosf)
    h1, ns, nd = _mm1(xp, W1, od, idg)
    p1 = _agg_call()(h1, srcc, dstc)
    h2 = _mm2(p1, nd, b1, ns, W2)
    p2 = _agg_call()(h2, srcc, dstc)
    out = _mm3(p2, nd, b2)
    return out[:_N]


# speedup vs baseline: 22.5823x; 22.5823x over previous
import functools

import jax
import jax.numpy as jnp
from jax import lax
from jax.experimental import pallas as pl
from jax.experimental.pallas import tpu as pltpu
from jax.experimental.pallas import tpu_sc as plsc

_N = 10000
_E = 320000
_D = 128

_NC = 2
_NS = 16

_CH = 128
_C = 160
_EPAD = _NS * _C * _CH
_NBUF = 2
_SS = 2
_SW = _CH // _SS
_RB = 128
_NPAD = 10240
_NH = _NPAD // _NC
_NHB = _NH // _RB
_AROWS = _NH
_DSPLIT = (_NHB - 1) * _RB


def _deg_body(idxc, ones_hbm, zeros_hbm, deg_out,
              idx_v, ones_v, zeros_v, acc):
    cid = lax.axis_index("c")
    sid = lax.axis_index("s")

    pltpu.sync_copy(ones_hbm, ones_v)
    pltpu.sync_copy(zeros_hbm, zeros_v)

    def zloop(k, carry):
        b = sid + _NS * k

        @pl.when(b < _NHB)
        def _z():
            pltpu.sync_copy(zeros_v, acc.at[pl.ds(b * _CH, _CH)])
        return carry

    lax.fori_loop(0, _NHB // _NS + 1, zloop, 0)
    pltpu.sync_copy(idxc.at[cid].at[sid], idx_v)
    plsc.subcore_barrier()

    def eloop(j, carry):
        pltpu.sync_copy(ones_v, acc.at[idx_v.at[j]], add=True)
        return carry

    lax.fori_loop(0, _C, eloop, 0)
    plsc.subcore_barrier()

    def oloop(k, carry):
        b = sid + _NS * k

        @pl.when(b < _NHB)
        def _o():
            pltpu.sync_copy(acc.at[pl.ds(b * _CH, _CH)],
                            deg_out.at[cid].at[pl.ds(b * _CH, _CH)])
        return carry

    lax.fori_loop(0, _NHB // _NS + 1, oloop, 0)


@functools.cache
def _deg_call():
    mesh = plsc.VectorSubcoreMesh(
        core_axis_name="c", subcore_axis_name="s",
        num_cores=_NC, num_subcores=_NS)
    return pl.kernel(
        _deg_body,
        out_type=jax.ShapeDtypeStruct((_NC, _NH, _D), jnp.float32),
        mesh=mesh,
        scratch_types=[
            pltpu.VMEM((_C, _CH), jnp.int32),
            pltpu.VMEM((_CH, _D), jnp.float32),
            pltpu.VMEM((_CH, _D), jnp.float32),
            pltpu.VMEM_SHARED((_NH, _D), jnp.float32),
        ],
    )


def _agg_body(h_hbm, srcb, dstc, out_hbm,
              src_v, dst_v, rows_v, acc, *sems):
    cid = lax.axis_index("c")
    sid = lax.axis_index("s")

    pltpu.sync_copy(h_hbm.at[pl.ds(_NPAD - _RB, _RB)], rows_v.at[0])

    def zloop(k, carry):
        b = sid + _NS * k

        @pl.when(b < _NHB)
        def _z():
            pltpu.sync_copy(rows_v.at[0], acc.at[pl.ds(b * _RB, _RB)])
        return carry

    lax.fori_loop(0, _NHB // _NS + 1, zloop, 0)
    pltpu.sync_copy(srcb.at[cid].at[sid], src_v)
    pltpu.sync_copy(dstc.at[cid].at[sid], dst_v)
    plsc.subcore_barrier()

    def _fire(j, slot, sem):
        for s in range(_SS):
            pltpu.async_copy(
                h_hbm.at[src_v.at[j].at[pl.ds(s * _SW, _SW)]],
                rows_v.at[slot].at[pl.ds(s * _SW, _SW)], sem)

    def _drain(slot, sem):
        pltpu.make_async_copy(
            h_hbm.at[pl.ds(0, _CH)], rows_v.at[slot], sem).wait()

    for r in range(_NBUF):
        _fire(r, r, sems[r])

    def gloop(g, carry):
        for r in range(_NBUF):
            j = _NBUF * g + r
            _drain(r, sems[r])
            pltpu.sync_copy(rows_v.at[r], acc.at[dst_v.at[j]], add=True)

            @pl.when(j + _NBUF < _C)
            def _next():
                _fire(j + _NBUF, r, sems[r])
        return carry

    lax.fori_loop(0, _C // _NBUF, gloop, 0)
    plsc.subcore_barrier()

    def oloop(k, carry):
        b = sid + _NS * k

        @pl.when(b < _NHB)
        def _o():
            pltpu.sync_copy(acc.at[pl.ds(b * _CH, _CH)],
                            out_hbm.at[cid].at[pl.ds(b * _CH, _CH)])
        return carry

    lax.fori_loop(0, _NHB // _NS + 1, oloop, 0)


@functools.cache
def _agg_call():
    mesh = plsc.VectorSubcoreMesh(
        core_axis_name="c", subcore_axis_name="s",
        num_cores=_NC, num_subcores=_NS)
    return pl.kernel(
        _agg_body,
        out_type=jax.ShapeDtypeStruct((_NC, _NH, _D), jnp.float32),
        mesh=mesh,
        scratch_types=[
            pltpu.VMEM((_C, _CH), jnp.int32),
            pltpu.VMEM((_C, _CH), jnp.int32),
            pltpu.VMEM((_NBUF, _CH, _D), jnp.float32),
            pltpu.VMEM_SHARED((_AROWS, _D), jnp.float32),
        ] + [pltpu.SemaphoreType.DMA] * _NBUF,
    )


def _deg_block(i):
    c = jnp.where(i < _NHB - 1, 1, 0)
    r = jnp.where(i < _NHB - 1, i, jnp.minimum(i - (_NHB - 1), _NHB - 1))
    return (c, r, 0)


def _mm1_body(x_ref, w_ref, od_ref, id_ref, h_ref, ns_ref, nd_ref):
    odeg = od_ref[0][:, :1]
    ideg = id_ref[0][:, :1]
    ns = lax.rsqrt(jnp.maximum(odeg, 1.0))
    nd = lax.rsqrt(jnp.maximum(ideg, 1.0))
    h = jnp.dot(x_ref[...], w_ref[...], preferred_element_type=jnp.float32)
    h_ref[...] = h * ns
    ns_ref[...] = ns
    nd_ref[...] = nd


_mm1 = pl.pallas_call(
    _mm1_body,
    grid=(_NPAD // _RB,),
    in_specs=[
        pl.BlockSpec((_RB, _D), lambda i: (i, 0)),
        pl.BlockSpec((_D, _D), lambda i: (0, 0)),
        pl.BlockSpec((1, _RB, _D), _deg_block),
        pl.BlockSpec((1, _RB, _D), _deg_block),
    ],
    out_specs=[
        pl.BlockSpec((_RB, _D), lambda i: (i, 0)),
        pl.BlockSpec((_RB, 1), lambda i: (i, 0)),
        pl.BlockSpec((_RB, 1), lambda i: (i, 0)),
    ],
    out_shape=[
        jax.ShapeDtypeStruct((_NPAD, _D), jnp.float32),
        jax.ShapeDtypeStruct((_NPAD, 1), jnp.float32),
        jax.ShapeDtypeStruct((_NPAD, 1), jnp.float32),
    ],
)


def _mm2_body(p_ref, nd_ref, b1_ref, ns_ref, w_ref, out_ref):
    i = pl.program_id(0)
    agg = p_ref[0]
    h = jnp.tanh(agg * nd_ref[...] + b1_ref[...])
    rows = i * _RB + lax.broadcasted_iota(jnp.int32, (_RB, 1), 0)
    h = jnp.where(rows < _N, h, 0.0)
    out_ref[...] = jnp.dot(h * ns_ref[...], w_ref[...],
                           preferred_element_type=jnp.float32)


_mm2 = pl.pallas_call(
    _mm2_body,
    grid=(_NPAD // _RB,),
    in_specs=[
        pl.BlockSpec((1, _RB, _D), lambda i: (i // _NHB, i % _NHB, 0)),
        pl.BlockSpec((_RB, 1), lambda i: (i, 0)),
        pl.BlockSpec((_D,), lambda i: (0,)),
        pl.BlockSpec((_RB, 1), lambda i: (i, 0)),
        pl.BlockSpec((_D, _D), lambda i: (0, 0)),
    ],
    out_specs=pl.BlockSpec((_RB, _D), lambda i: (i, 0)),
    out_shape=jax.ShapeDtypeStruct((_NPAD, _D), jnp.float32),
)


def _mm3_body(p_ref, nd_ref, b2_ref, out_ref):
    out_ref[...] = p_ref[0] * nd_ref[...] + b2_ref[...]


_mm3 = pl.pallas_call(
    _mm3_body,
    grid=(_NPAD // _RB,),
    in_specs=[
        pl.BlockSpec((1, _RB, _D), lambda i: (i // _NHB, i % _NHB, 0)),
        pl.BlockSpec((_RB, 1), lambda i: (i, 0)),
        pl.BlockSpec((_D,), lambda i: (0,)),
    ],
    out_specs=pl.BlockSpec((_RB, _D), lambda i: (i, 0)),
    out_shape=jax.ShapeDtypeStruct((_NPAD, _D), jnp.float32),
)


def kernel(graph, x, W1, b1, W2, b2):
    src = graph[0].astype(jnp.int32)
    dst = graph[1].astype(jnp.int32)
    zrow = _N + (jnp.arange(_EPAD, dtype=jnp.int32) % (_NPAD - _N))
    padv = zrow[_E:]
    dstp = jnp.concatenate([dst, jnp.full((_EPAD - _E,), _N, jnp.int32)])
    srcp = jnp.concatenate([src, padv])

    def _local(idx, c):
        return idx - c * _NH

    def _owned(idx, c):
        loc = _local(idx, c)
        return (loc >= 0) & (loc < _NH)

    src_halves, dst_halves = [], []
    for c in range(_NC):
        own_d = _owned(dstp, c)
        src_halves.append(
            jnp.where(own_d, srcp, zrow).reshape(_NS, _C, _CH))
        dst_halves.append(
            jnp.where(own_d, _local(dstp, c),
                      zrow - _N).reshape(_NS, _C, _CH))
    srcc = jnp.stack(src_halves)
    dstc = jnp.stack(dst_halves)

    def _deg_idx(idx):
        c0 = jnp.where(idx >= _DSPLIT, idx - _DSPLIT, _NH - 1)
        c1 = jnp.where(idx < _DSPLIT, idx, _DSPLIT)
        return jnp.stack([c0.reshape(_NS, _C, _CH),
                          c1.reshape(_NS, _C, _CH)])

    sdegc = _deg_idx(srcp)
    ddegc = _deg_idx(dstp)
    xp = jnp.pad(x, ((0, _NPAD - _N), (0, 0)))
    onesf = jnp.ones((_CH, _D), jnp.float32)
    zerosf = jnp.zeros((_CH, _D), jnp.float32)

    od = _deg_call()(sdegc, onesf, zerosf)
    idg = _deg_call()(ddegc, onesf, zerosf)
    h1, ns, nd = _mm1(xp, W1, od, idg)
    p1 = _agg_call()(h1, srcc, dstc)
    h2 = _mm2(p1, nd, b1, ns, W2)
    p2 = _agg_call()(h2, srcc, dstc)
    out = _mm3(p2, nd, b2)
    return out[:_N]


# speedup vs baseline: 26.5656x; 1.1764x over previous
import functools

import jax
import jax.numpy as jnp
from jax import lax
from jax.experimental import pallas as pl
from jax.experimental.pallas import tpu as pltpu
from jax.experimental.pallas import tpu_sc as plsc

_N = 10000
_E = 320000
_D = 128

_NC = 2
_NS = 16

_CH = 128
_C = 160
_EPAD = _NS * _C * _CH
_NBUF = 2
_SS = 2
_SW = _CH // _SS
_RB = 128
_NPAD = 10240
_NH = _NPAD // _NC
_NHB = _NH // _RB
_AROWS = _NH
_DSPLIT = (_NHB - 1) * _RB


def _deg_body(idxc, ones_hbm, zeros_hbm, deg_out,
              idx_v, ones_v, zeros_v, acc):
    cid = lax.axis_index("c")
    sid = lax.axis_index("s")

    pltpu.sync_copy(ones_hbm, ones_v)
    pltpu.sync_copy(zeros_hbm, zeros_v)

    def zloop(k, carry):
        b = sid + _NS * k

        @pl.when(b < _NHB)
        def _z():
            pltpu.sync_copy(zeros_v, acc.at[pl.ds(b * _CH, _CH)])
        return carry

    lax.fori_loop(0, _NHB // _NS + 1, zloop, 0)
    pltpu.sync_copy(idxc.at[cid].at[sid], idx_v)
    plsc.subcore_barrier()

    def eloop(j, carry):
        pltpu.sync_copy(ones_v, acc.at[idx_v.at[j]], add=True)
        return carry

    lax.fori_loop(0, _C, eloop, 0)
    plsc.subcore_barrier()

    def oloop(k, carry):
        b = sid + _NS * k

        @pl.when(b < _NHB)
        def _o():
            pltpu.sync_copy(acc.at[pl.ds(b * _CH, _CH)],
                            deg_out.at[cid].at[pl.ds(b * _CH, _CH)])
        return carry

    lax.fori_loop(0, _NHB // _NS + 1, oloop, 0)


@functools.cache
def _deg_call():
    mesh = plsc.VectorSubcoreMesh(
        core_axis_name="c", subcore_axis_name="s",
        num_cores=_NC, num_subcores=_NS)
    return pl.kernel(
        _deg_body,
        out_type=jax.ShapeDtypeStruct((_NC, _NH, _D), jnp.float32),
        mesh=mesh,
        scratch_types=[
            pltpu.VMEM((_C, _CH), jnp.int32),
            pltpu.VMEM((_CH, _D), jnp.float32),
            pltpu.VMEM((_CH, _D), jnp.float32),
            pltpu.VMEM_SHARED((_NH, _D), jnp.float32),
        ],
    )


def _agg_body(h_hbm, srcb, dstc, out_hbm,
              src_v, dst_v, rows_v, acc, *sems):
    cid = lax.axis_index("c")
    sid = lax.axis_index("s")

    pltpu.sync_copy(h_hbm.at[pl.ds(_NPAD - _RB, _RB)], rows_v.at[0])

    def zloop(k, carry):
        b = sid + _NS * k

        @pl.when(b < _NHB)
        def _z():
            pltpu.sync_copy(rows_v.at[0], acc.at[pl.ds(b * _RB, _RB)])
        return carry

    lax.fori_loop(0, _NHB // _NS + 1, zloop, 0)
    pltpu.sync_copy(srcb.at[cid].at[sid], src_v)
    pltpu.sync_copy(dstc.at[cid].at[sid], dst_v)
    plsc.subcore_barrier()

    def _fire(j, slot, sem):
        for s in range(_SS):
            pltpu.async_copy(
                h_hbm.at[src_v.at[j].at[pl.ds(s * _SW, _SW)]],
                rows_v.at[slot].at[pl.ds(s * _SW, _SW)], sem)

    def _drain(slot, sem):
        pltpu.make_async_copy(
            h_hbm.at[pl.ds(0, _CH)], rows_v.at[slot], sem).wait()

    for r in range(_NBUF):
        _fire(r, r, sems[r])

    def gloop(g, carry):
        for r in range(_NBUF):
            j = _NBUF * g + r
            _drain(r, sems[r])
            pltpu.sync_copy(rows_v.at[r], acc.at[dst_v.at[j]], add=True)

            @pl.when(j + _NBUF < _C)
            def _next():
                _fire(j + _NBUF, r, sems[r])
        return carry

    lax.fori_loop(0, _C // _NBUF, gloop, 0)
    plsc.subcore_barrier()

    def oloop(k, carry):
        b = sid + _NS * k

        @pl.when(b < _NHB)
        def _o():
            pltpu.sync_copy(acc.at[pl.ds(b * _CH, _CH)],
                            out_hbm.at[cid].at[pl.ds(b * _CH, _CH)])
        return carry

    lax.fori_loop(0, _NHB // _NS + 1, oloop, 0)


@functools.cache
def _agg_call():
    mesh = plsc.VectorSubcoreMesh(
        core_axis_name="c", subcore_axis_name="s",
        num_cores=_NC, num_subcores=_NS)
    return pl.kernel(
        _agg_body,
        out_type=jax.ShapeDtypeStruct((_NC, _NH, _D), jnp.float32),
        mesh=mesh,
        scratch_types=[
            pltpu.VMEM((_C, _CH), jnp.int32),
            pltpu.VMEM((_C, _CH), jnp.int32),
            pltpu.VMEM((_NBUF, _CH, _D), jnp.float32),
            pltpu.VMEM_SHARED((_AROWS, _D), jnp.float32),
        ] + [pltpu.SemaphoreType.DMA] * _NBUF,
    )


def _deg_block(i):
    c = jnp.where(i < _NHB - 1, 1, 0)
    r = jnp.where(i < _NHB - 1, i, jnp.minimum(i - (_NHB - 1), _NHB - 1))
    return (c, r, 0)


def _mm1_body(x_ref, w_ref, od_ref, id_ref, h_ref, ns_ref, nd_ref):
    odeg = od_ref[0][:, :1]
    ideg = id_ref[0][:, :1]
    ns = lax.rsqrt(jnp.maximum(odeg, 1.0))
    nd = lax.rsqrt(jnp.maximum(ideg, 1.0))
    h = jnp.dot(x_ref[...], w_ref[...], preferred_element_type=jnp.float32)
    h_ref[...] = h * ns
    ns_ref[...] = ns
    nd_ref[...] = nd


_mm1 = pl.pallas_call(
    _mm1_body,
    grid=(_NPAD // _RB,),
    in_specs=[
        pl.BlockSpec((_RB, _D), lambda i: (i, 0)),
        pl.BlockSpec((_D, _D), lambda i: (0, 0)),
        pl.BlockSpec((1, _RB, _D), _deg_block),
        pl.BlockSpec((1, _RB, _D), _deg_block),
    ],
    out_specs=[
        pl.BlockSpec((_RB, _D), lambda i: (i, 0)),
        pl.BlockSpec((_RB, 1), lambda i: (i, 0)),
        pl.BlockSpec((_RB, 1), lambda i: (i, 0)),
    ],
    out_shape=[
        jax.ShapeDtypeStruct((_NPAD, _D), jnp.float32),
        jax.ShapeDtypeStruct((_NPAD, 1), jnp.float32),
        jax.ShapeDtypeStruct((_NPAD, 1), jnp.float32),
    ],
)


def _mm2_body(p_ref, nd_ref, b1_ref, ns_ref, w_ref, out_ref):
    i = pl.program_id(0)
    agg = p_ref[0]
    h = jnp.tanh(agg * nd_ref[...] + b1_ref[...])
    rows = i * _RB + lax.broadcasted_iota(jnp.int32, (_RB, 1), 0)
    h = jnp.where(rows < _N, h, 0.0)
    out_ref[...] = jnp.dot(h * ns_ref[...], w_ref[...],
                           preferred_element_type=jnp.float32)


_mm2 = pl.pallas_call(
    _mm2_body,
    grid=(_NPAD // _RB,),
    in_specs=[
        pl.BlockSpec((1, _RB, _D), lambda i: (i // _NHB, i % _NHB, 0)),
        pl.BlockSpec((_RB, 1), lambda i: (i, 0)),
        pl.BlockSpec((_D,), lambda i: (0,)),
        pl.BlockSpec((_RB, 1), lambda i: (i, 0)),
        pl.BlockSpec((_D, _D), lambda i: (0, 0)),
    ],
    out_specs=pl.BlockSpec((_RB, _D), lambda i: (i, 0)),
    out_shape=jax.ShapeDtypeStruct((_NPAD, _D), jnp.float32),
)


def _mm3_body(p_ref, nd_ref, b2_ref, out_ref):
    out_ref[...] = p_ref[0] * nd_ref[...] + b2_ref[...]


_mm3 = pl.pallas_call(
    _mm3_body,
    grid=(_NPAD // _RB,),
    in_specs=[
        pl.BlockSpec((1, _RB, _D), lambda i: (i // _NHB, i % _NHB, 0)),
        pl.BlockSpec((_RB, 1), lambda i: (i, 0)),
        pl.BlockSpec((_D,), lambda i: (0,)),
    ],
    out_specs=pl.BlockSpec((_RB, _D), lambda i: (i, 0)),
    out_shape=jax.ShapeDtypeStruct((_NPAD, _D), jnp.float32),
)


def kernel(graph, x, W1, b1, W2, b2):
    src = graph[0].astype(jnp.int32)
    dst = graph[1].astype(jnp.int32)
    zrow = _N + (jnp.arange(_EPAD, dtype=jnp.int32) % (_NPAD - _N))
    padv = zrow[_E:]
    dstp = jnp.concatenate([dst, jnp.full((_EPAD - _E,), _N, jnp.int32)])
    srcp = jnp.concatenate([src, padv])

    def _local(idx, c):
        return idx - c * _NH

    def _owned(idx, c):
        loc = _local(idx, c)
        return (loc >= 0) & (loc < _NH)

    src_halves, dst_halves = [], []
    for c in range(_NC):
        own_d = _owned(dstp, c)
        src_halves.append(
            jnp.where(own_d, srcp, zrow).reshape(_NS, _C, _CH))
        dst_halves.append(
            jnp.where(own_d, _local(dstp, c),
                      zrow - _N).reshape(_NS, _C, _CH))
    srcc = jnp.stack(src_halves)
    dstc = jnp.stack(dst_halves)

    pos = jnp.arange(_EPAD, dtype=jnp.int32)
    dump0 = 5012 + pos % (_NH - 5012)
    dump1 = _DSPLIT + pos % (_NH - _DSPLIT)

    def _deg_idx(idx):
        c0 = jnp.where(idx >= _DSPLIT, idx - _DSPLIT, dump0)
        c1 = jnp.where(idx < _DSPLIT, idx, dump1)
        return jnp.stack([c0.reshape(_NS, _C, _CH),
                          c1.reshape(_NS, _C, _CH)])

    sdegc = _deg_idx(srcp)
    ddegc = _deg_idx(dstp)
    xp = jnp.pad(x, ((0, _NPAD - _N), (0, 0)))
    onesf = jnp.ones((_CH, _D), jnp.float32)
    zerosf = jnp.zeros((_CH, _D), jnp.float32)

    od = _deg_call()(sdegc, onesf, zerosf)
    idg = _deg_call()(ddegc, onesf, zerosf)
    h1, ns, nd = _mm1(xp, W1, od, idg)
    p1 = _agg_call()(h1, srcc, dstc)
    h2 = _mm2(p1, nd, b1, ns, W2)
    p2 = _agg_call()(h2, srcc, dstc)
    out = _mm3(p2, nd, b2)
    return out[:_N]
